# baseline (speedup 1.0000x reference)
import jax
import jax.numpy as jnp
from jax.experimental import pallas as pl


def _gatv2(x, edge_index, Wl, Wr, att, b):
    n = x.shape[0]
    loops = jnp.arange(n, dtype=edge_index.dtype)
    src = jnp.concatenate([edge_index[0], loops])
    dst = jnp.concatenate([edge_index[1], loops])
    xl = x @ Wl
    xr = x @ Wr
    m = jax.nn.leaky_relu(xl[src] + xr[dst], negative_slope=0.2)
    alpha = m @ att
    amax = jax.ops.segment_max(alpha, dst, num_segments=n)
    e = jnp.exp(alpha - amax[dst])
    denom = jax.ops.segment_sum(e, dst, num_segments=n)
    w = e / (denom[dst] + 1e-16)
    out = jax.ops.segment_sum(xl[src] * w[:, None], dst, num_segments=n)
    return out + b


def _reparam_kernel(mu_ref, ls_ref, eps_ref, z_ref):
    z_ref[...] = eps_ref[...] * jnp.exp(ls_ref[...]) + mu_ref[...]


def kernel(x, edge_index, c1_Wl, c1_Wr, c1_att, c1_b, c2_Wl, c2_Wr, c2_att, c2_b, mu_Wl, mu_Wr, mu_att, mu_b, ls_Wl, ls_Wr, ls_att, ls_b):
    h = jax.nn.elu(_gatv2(x, edge_index, c1_Wl, c1_Wr, c1_att, c1_b))
    h = jax.nn.elu(_gatv2(h, edge_index, c2_Wl, c2_Wr, c2_att, c2_b))
    mu = _gatv2(h, edge_index, mu_Wl, mu_Wr, mu_att, mu_b)
    logstd = _gatv2(h, edge_index, ls_Wl, ls_Wr, ls_att, ls_b)
    eps = jax.random.normal(jax.random.key(42), logstd.shape, dtype=logstd.dtype)
    z = pl.pallas_call(
        _reparam_kernel,
        out_shape=jax.ShapeDtypeStruct(mu.shape, mu.dtype),
    )(mu, logstd, eps)
    return (mu, logstd, z)


# speedup vs baseline: 8.0174x; 8.0174x over previous
import jax
import jax.numpy as jnp
from jax import lax
from jax.experimental import pallas as pl
from jax.experimental.pallas import tpu as pltpu
from jax.experimental.pallas import tpu_sc as plsc

NC = 2
NS = 16
L = 16
NW = NC * NS
CH = 128
SH_ROWS = 10240
RB = 512


def _vgather(x, idx):
    dnums = lax.GatherDimensionNumbers(
        offset_dims=(), collapsed_slice_dims=(0,), start_index_map=(0,))
    return lax.gather(x, idx[:, None], dnums, slice_sizes=(1,),
                      mode=lax.GatherScatterMode.PROMISE_IN_BOUNDS)


def _edge_sc(xl, xr, src, dst, att6, att4, heads):
    D = xl.shape[1]
    ch = 64 if D > 64 else CH
    kh = D // heads // L
    e_pad = src.shape[0]
    ept = e_pad // NW
    nchunk = ept // ch
    denw = heads * SH_ROWS
    rpt = SH_ROWS // NS
    ncp = rpt // ch
    mesh = plsc.VectorSubcoreMesh(core_axis_name="c", subcore_axis_name="s")

    def body(xl_hbm, xr_hbm, src_hbm, dst_hbm, a6_hbm, a4_hbm,
             out_hbm, den_hbm,
             acc_sh, gl, gr, prow, srcv, dstv, denv, a6v, a4v, sem1, sem2):
        cid = lax.axis_index("c")
        sid = lax.axis_index("s")
        wid = cid * NS + sid
        zero = jnp.zeros((L,), jnp.float32)

        def zrow(r, _):
            for k in range(D // L):
                prow[r, pl.ds(k * L, L)] = zero
            return 0
        lax.fori_loop(0, ch, zrow, 0)

        def zden(i, _):
            denv[pl.ds(i * L, L)] = zero
            return 0
        lax.fori_loop(0, denw // L, zden, 0)

        for j in range(ncp):
            pltpu.sync_copy(prow, acc_sh.at[pl.ds(sid * rpt + j * ch, ch)])
        pltpu.sync_copy(a6_hbm, a6v)
        pltpu.sync_copy(a4_hbm, a4v)
        plsc.subcore_barrier()

        a6r = [a6v[pl.ds(k * L, L)] for k in range(D // L)]
        a4r = [a4v[pl.ds(k * L, L)] for k in range(D // L)]
        iota = lax.iota(jnp.int32, L)
        shifts = [(iota + s) & (L - 1) for s in (8, 4, 2, 1)]

        def chunk_body(c, _):
            base = wid * ept + c * ch
            pltpu.sync_copy(src_hbm.at[pl.ds(base, ch)], srcv)
            pltpu.sync_copy(dst_hbm.at[pl.ds(base, ch)], dstv)
            cp1 = pltpu.async_copy(xl_hbm.at[srcv], gl, sem1)
            cp2 = pltpu.async_copy(xr_hbm.at[dstv], gr, sem2)
            cp1.wait()
            cp2.wait()

            def group_body(g, _):
                def edge_body(j, eaccs):
                    e_i = g * L + j
                    new = []
                    for h in range(heads):
                        acc = zero
                        for k in range(kh):
                            col = (h * kh + k) * L
                            va = gl[e_i, pl.ds(col, L)]
                            vb = gr[e_i, pl.ds(col, L)]
                            z = va + vb
                            acc = acc + a6r[h * kh + k] * z \
                                + a4r[h * kh + k] * jnp.abs(z)
                        for sh in shifts:
                            acc = acc + _vgather(acc, sh)
                        ev = jnp.exp(acc)
                        for k in range(kh):
                            col = (h * kh + k) * L
                            prow[e_i, pl.ds(col, L)] = \
                                gl[e_i, pl.ds(col, L)] * ev
                        new.append(jnp.where(iota == j, ev, eaccs[h]))
                    return tuple(new)
                eaccs = lax.fori_loop(0, L, edge_body,
                                      tuple(zero for _ in range(heads)))
                d16 = dstv[pl.ds(g * L, L)]
                for h in range(heads):
                    plsc.addupdate_scatter(denv, [d16 + h * SH_ROWS], eaccs[h])
                return 0
            lax.fori_loop(0, ch // L, group_body, 0)
            pltpu.sync_copy(prow, acc_sh.at[dstv], add=True)
            return 0
        lax.fori_loop(0, nchunk, chunk_body, 0)

        plsc.subcore_barrier()
        for j in range(ncp):
            r0 = sid * rpt + j * ch
            pltpu.sync_copy(acc_sh.at[pl.ds(r0, ch)],
                            out_hbm.at[cid, pl.ds(r0, ch)])
        pltpu.sync_copy(denv, den_hbm.at[wid])

    f = pl.kernel(
        body,
        out_type=(jax.ShapeDtypeStruct((NC, SH_ROWS, D), jnp.float32),
                  jax.ShapeDtypeStruct((NW, denw), jnp.float32)),
        mesh=mesh,
        compiler_params=pltpu.CompilerParams(use_tc_tiling_on_sc=False,
                                             needs_layout_passes=False),
        scratch_types=[
            pltpu.VMEM_SHARED((SH_ROWS, D), jnp.float32),
            pltpu.VMEM((ch, D), jnp.float32),
            pltpu.VMEM((ch, D), jnp.float32),
            pltpu.VMEM((ch, D), jnp.float32),
            pltpu.VMEM((ch,), jnp.int32),
            pltpu.VMEM((ch,), jnp.int32),
            pltpu.VMEM((denw,), jnp.float32),
            pltpu.VMEM((D,), jnp.float32),
            pltpu.VMEM((D,), jnp.float32),
            pltpu.SemaphoreType.DMA,
            pltpu.SemaphoreType.DMA,
        ],
    )
    return f(xl, xr, src, dst, att6, att4)


def _proj_tc(h, Wl, Wr):
    din, dl = Wl.shape
    dr = Wr.shape[1]

    def body(h_ref, wl_ref, wr_ref, xl_ref, xr_ref):
        hv = h_ref[...]
        xl_ref[...] = jnp.dot(hv, wl_ref[...],
                              preferred_element_type=jnp.float32)
        xr_ref[...] = jnp.dot(hv, wr_ref[...],
                              preferred_element_type=jnp.float32)

    return pl.pallas_call(
        body,
        grid=(SH_ROWS // RB,),
        in_specs=[pl.BlockSpec((RB, din), lambda i: (i, 0)),
                  pl.BlockSpec((din, dl), lambda i: (0, 0)),
                  pl.BlockSpec((din, dr), lambda i: (0, 0))],
        out_specs=[pl.BlockSpec((RB, dl), lambda i: (i, 0)),
                   pl.BlockSpec((RB, dr), lambda i: (i, 0))],
        out_shape=[jax.ShapeDtypeStruct((SH_ROWS, dl), jnp.float32),
                   jax.ShapeDtypeStruct((SH_ROWS, dr), jnp.float32)],
    )(h, Wl, Wr)


def _merge_proj_tc(parts, den, b, Wl, Wr):
    D = parts.shape[2]
    dl = Wl.shape[1]
    dr = Wr.shape[1]

    def body(p_ref, den_ref, b_ref, wl_ref, wr_ref, xl_ref, xr_ref):
        p = p_ref[0] + p_ref[1]
        dsum = jnp.sum(den_ref[...], axis=0)
        hv = p * (1.0 / (dsum + 1e-16))[:, None] + b_ref[...][None, :]
        hv = jnp.where(hv > 0, hv, jnp.exp(jnp.minimum(hv, 0.0)) - 1.0)
        xl_ref[...] = jnp.dot(hv, wl_ref[...],
                              preferred_element_type=jnp.float32)
        xr_ref[...] = jnp.dot(hv, wr_ref[...],
                              preferred_element_type=jnp.float32)

    return pl.pallas_call(
        body,
        grid=(SH_ROWS // RB,),
        in_specs=[pl.BlockSpec((NC, RB, D), lambda i: (0, i, 0)),
                  pl.BlockSpec((NW, RB), lambda i: (0, i)),
                  pl.BlockSpec((D,), lambda i: (0,)),
                  pl.BlockSpec((D, dl), lambda i: (0, 0)),
                  pl.BlockSpec((D, dr), lambda i: (0, 0))],
        out_specs=[pl.BlockSpec((RB, dl), lambda i: (i, 0)),
                   pl.BlockSpec((RB, dr), lambda i: (i, 0))],
        out_shape=[jax.ShapeDtypeStruct((SH_ROWS, dl), jnp.float32),
                   jax.ShapeDtypeStruct((SH_ROWS, dr), jnp.float32)],
    )(parts, den, b, Wl, Wr)


def _final_tc(parts, den3, mu_b, ls_b, eps):
    lat = mu_b.shape[0]
    D = parts.shape[2]

    def body(p_ref, den_ref, mb_ref, lb_ref, eps_ref, mu_ref, ls_ref, z_ref):
        p = p_ref[0] + p_ref[1]
        ds = jnp.sum(den_ref[...], axis=0)
        mu = p[:, :lat] * (1.0 / (ds[0] + 1e-16))[:, None] \
            + mb_ref[...][None, :]
        ls = p[:, lat:] * (1.0 / (ds[1] + 1e-16))[:, None] \
            + lb_ref[...][None, :]
        mu_ref[...] = mu
        ls_ref[...] = ls
        z_ref[...] = eps_ref[...] * jnp.exp(ls) + mu

    return pl.pallas_call(
        body,
        grid=(SH_ROWS // RB,),
        in_specs=[pl.BlockSpec((NC, RB, D), lambda i: (0, i, 0)),
                  pl.BlockSpec((NW, 2, RB), lambda i: (0, 0, i)),
                  pl.BlockSpec((lat,), lambda i: (0,)),
                  pl.BlockSpec((lat,), lambda i: (0,)),
                  pl.BlockSpec((RB, lat), lambda i: (i, 0))],
        out_specs=[pl.BlockSpec((RB, lat), lambda i: (i, 0)),
                   pl.BlockSpec((RB, lat), lambda i: (i, 0)),
                   pl.BlockSpec((RB, lat), lambda i: (i, 0))],
        out_shape=[jax.ShapeDtypeStruct((SH_ROWS, lat), jnp.float32),
                   jax.ShapeDtypeStruct((SH_ROWS, lat), jnp.float32),
                   jax.ShapeDtypeStruct((SH_ROWS, lat), jnp.float32)],
    )(parts, den3, mu_b, ls_b, eps)


def kernel(x, edge_index, c1_Wl, c1_Wr, c1_att, c1_b, c2_Wl, c2_Wr, c2_att,
           c2_b, mu_Wl, mu_Wr, mu_att, mu_b, ls_Wl, ls_Wr, ls_att, ls_b):
    n = x.shape[0]
    e = edge_index.shape[1]
    loops = jnp.arange(n, dtype=jnp.int32)
    src = jnp.concatenate([edge_index[0], loops])
    dst = jnp.concatenate([edge_index[1], loops])
    e_real = e + n
    e_pad = ((e_real + NW * CH - 1) // (NW * CH)) * (NW * CH)
    src = jnp.concatenate(
        [src, jnp.zeros((e_pad - e_real,), jnp.int32)])
    dst = jnp.concatenate(
        [dst, jnp.full((e_pad - e_real,), n, jnp.int32)])

    xpad = jnp.pad(x, ((0, SH_ROWS - n), (0, 0)))

    xl1, xr1 = _proj_tc(xpad, c1_Wl, c1_Wr)
    p1, d1 = _edge_sc(xl1, xr1, src, dst, 0.6 * c1_att, 0.4 * c1_att, 1)
    xl2, xr2 = _merge_proj_tc(p1, d1, c1_b, c2_Wl, c2_Wr)
    p2, d2 = _edge_sc(xl2, xr2, src, dst, 0.6 * c2_att, 0.4 * c2_att, 1)
    Wl3 = jnp.concatenate([mu_Wl, ls_Wl], axis=1)
    Wr3 = jnp.concatenate([mu_Wr, ls_Wr], axis=1)
    a3 = jnp.concatenate([mu_att, ls_att])
    xl3, xr3 = _merge_proj_tc(p2, d2, c2_b, Wl3, Wr3)
    p3, d3 = _edge_sc(xl3, xr3, src, dst, 0.6 * a3, 0.4 * a3, 2)

    lat = mu_b.shape[0]
    eps = jax.random.normal(jax.random.key(42), (n, lat), jnp.float32)
    epad = jnp.pad(eps, ((0, SH_ROWS - n), (0, 0)))
    muf, lsf, zf = _final_tc(p3, d3.reshape(NW, 2, SH_ROWS),
                             mu_b, ls_b, epad)
    return muf[:n], lsf[:n], zf[:n]


# speedup vs baseline: 18.5661x; 2.3157x over previous
import jax
import jax.numpy as jnp
from jax import lax
from jax.experimental import pallas as pl
from jax.experimental.pallas import tpu as pltpu
from jax.experimental.pallas import tpu_sc as plsc

NC = 2
NS = 16
L = 16
NW = NC * NS
CH = 128
SH_ROWS = 10240
RB = 512


def _vgather(x, idx):
    dnums = lax.GatherDimensionNumbers(
        offset_dims=(), collapsed_slice_dims=(0,), start_index_map=(0,))
    return lax.gather(x, idx[:, None], dnums, slice_sizes=(1,),
                      mode=lax.GatherScatterMode.PROMISE_IN_BOUNDS)


def _edge_sc(xl, xr, sd, att6, att4, heads):
    D = xl.shape[1]
    ch = sd.shape[2]
    kh = D // heads // L
    nchunk = sd.shape[0] // NW
    denw = heads * SH_ROWS
    rpt = SH_ROWS // NS
    ncp = rpt // ch
    mesh = plsc.VectorSubcoreMesh(core_axis_name="c", subcore_axis_name="s")

    def body(xl_hbm, xr_hbm, sd_hbm, a6_hbm, a4_hbm,
             out_hbm, den_hbm,
             acc_sh, gl0, gl1, gr0, gr1, sdv, denv, a6v, a4v,
             sgl0, sgl1, sgr0, sgr1):
        cid = lax.axis_index("c")
        sid = lax.axis_index("s")
        wid = cid * NS + sid
        zero = jnp.zeros((L,), jnp.float32)
        gl = (gl0, gl1)
        gr = (gr0, gr1)
        sgl = (sgl0, sgl1)
        sgr = (sgr0, sgr1)

        def zrow(r, _):
            for k in range(D // L):
                gl0[r, pl.ds(k * L, L)] = zero
            return 0
        lax.fori_loop(0, ch, zrow, 0)

        def zden(i, _):
            denv[pl.ds(i * L, L)] = zero
            return 0
        lax.fori_loop(0, denw // L, zden, 0)

        for j in range(ncp):
            pltpu.sync_copy(gl0, acc_sh.at[pl.ds(sid * rpt + j * ch, ch)])
        pltpu.sync_copy(a6_hbm, a6v)
        pltpu.sync_copy(a4_hbm, a4v)
        plsc.subcore_barrier()

        a6r = [a6v[pl.ds(k * L, L)] for k in range(D // L)]
        a4r = [a4v[pl.ds(k * L, L)] for k in range(D // L)]
        iota = lax.iota(jnp.int32, L)
        shifts = [(iota + s) & (L - 1) for s in (8, 4, 2, 1)]

        def issue(g_idx, b):
            pltpu.sync_copy(sd_hbm.at[g_idx], sdv.at[b])
            pltpu.async_copy(xl_hbm.at[sdv.at[b, 0]], gl[b], sgl[b])
            pltpu.async_copy(xr_hbm.at[sdv.at[b, 1]], gr[b], sgr[b])

        issue(wid * nchunk, 0)

        def half_body(cc, b):
            c = cc * 2 + b
            pltpu.make_async_copy(xl_hbm.at[sdv.at[b, 0]], gl[b],
                                  sgl[b]).wait()
            pltpu.make_async_copy(xr_hbm.at[sdv.at[b, 1]], gr[b],
                                  sgr[b]).wait()

            @pl.when(c + 1 < nchunk)
            def _():
                issue(wid * nchunk + c + 1, 1 - b)

            def group_body(g, _):
                e16s = [zero for _ in range(heads)]
                for j in range(L):
                    e_i = g * L + j
                    for h in range(heads):
                        acc = None
                        for k in range(kh):
                            col = (h * kh + k) * L
                            z = gl[b][e_i, pl.ds(col, L)] \
                                + gr[b][e_i, pl.ds(col, L)]
                            t = a6r[h * kh + k] * z \
                                + a4r[h * kh + k] * jnp.abs(z)
                            acc = t if acc is None else acc + t
                        for sh in shifts:
                            acc = acc + _vgather(acc, sh)
                        ev = jnp.exp(acc)
                        for k in range(kh):
                            col = (h * kh + k) * L
                            gl[b][e_i, pl.ds(col, L)] = \
                                gl[b][e_i, pl.ds(col, L)] * ev
                        e16s[h] = jnp.where(iota == j, ev, e16s[h])
                d16 = sdv[b, 1, pl.ds(g * L, L)]
                for h in range(heads):
                    plsc.addupdate_scatter(denv, [d16 + h * SH_ROWS],
                                           e16s[h])
                return 0
            lax.fori_loop(0, ch // L, group_body, 0)
            pltpu.sync_copy(gl[b], acc_sh.at[sdv.at[b, 1]], add=True)

        def chunk_body(cc, _):
            half_body(cc, 0)
            half_body(cc, 1)
            return 0
        lax.fori_loop(0, nchunk // 2, chunk_body, 0)

        plsc.subcore_barrier()
        for j in range(ncp):
            r0 = sid * rpt + j * ch
            pltpu.sync_copy(acc_sh.at[pl.ds(r0, ch)],
                            out_hbm.at[cid, pl.ds(r0, ch)])
        pltpu.sync_copy(denv, den_hbm.at[wid])

    f = pl.kernel(
        body,
        out_type=(jax.ShapeDtypeStruct((NC, SH_ROWS, D), jnp.float32),
                  jax.ShapeDtypeStruct((NW, denw), jnp.float32)),
        mesh=mesh,
        compiler_params=pltpu.CompilerParams(use_tc_tiling_on_sc=False,
                                             needs_layout_passes=False),
        scratch_types=[
            pltpu.VMEM_SHARED((SH_ROWS, D), jnp.float32),
            pltpu.VMEM((ch, D), jnp.float32),
            pltpu.VMEM((ch, D), jnp.float32),
            pltpu.VMEM((ch, D), jnp.float32),
            pltpu.VMEM((ch, D), jnp.float32),
            pltpu.VMEM((2, 2, ch), jnp.int32),
            pltpu.VMEM((denw,), jnp.float32),
            pltpu.VMEM((D,), jnp.float32),
            pltpu.VMEM((D,), jnp.float32),
            pltpu.SemaphoreType.DMA,
            pltpu.SemaphoreType.DMA,
            pltpu.SemaphoreType.DMA,
            pltpu.SemaphoreType.DMA,
        ],
    )
    return f(xl, xr, sd, att6, att4)


def _proj_tc(h, Wl, Wr):
    din, dl = Wl.shape
    dr = Wr.shape[1]

    def body(h_ref, wl_ref, wr_ref, xl_ref, xr_ref):
        hv = h_ref[...]
        xl_ref[...] = jnp.dot(hv, wl_ref[...],
                              preferred_element_type=jnp.float32)
        xr_ref[...] = jnp.dot(hv, wr_ref[...],
                              preferred_element_type=jnp.float32)

    return pl.pallas_call(
        body,
        grid=(SH_ROWS // RB,),
        in_specs=[pl.BlockSpec((RB, din), lambda i: (i, 0)),
                  pl.BlockSpec((din, dl), lambda i: (0, 0)),
                  pl.BlockSpec((din, dr), lambda i: (0, 0))],
        out_specs=[pl.BlockSpec((RB, dl), lambda i: (i, 0)),
                   pl.BlockSpec((RB, dr), lambda i: (i, 0))],
        out_shape=[jax.ShapeDtypeStruct((SH_ROWS, dl), jnp.float32),
                   jax.ShapeDtypeStruct((SH_ROWS, dr), jnp.float32)],
    )(h, Wl, Wr)


def _merge_proj_tc(parts, den, b, Wl, Wr):
    D = parts.shape[2]
    dl = Wl.shape[1]
    dr = Wr.shape[1]

    def body(p_ref, den_ref, b_ref, wl_ref, wr_ref, xl_ref, xr_ref):
        p = p_ref[0] + p_ref[1]
        dsum = jnp.sum(den_ref[...], axis=0)
        hv = p * (1.0 / (dsum + 1e-16))[:, None] + b_ref[...][None, :]
        hv = jnp.where(hv > 0, hv, jnp.exp(jnp.minimum(hv, 0.0)) - 1.0)
        xl_ref[...] = jnp.dot(hv, wl_ref[...],
                              preferred_element_type=jnp.float32)
        xr_ref[...] = jnp.dot(hv, wr_ref[...],
                              preferred_element_type=jnp.float32)

    return pl.pallas_call(
        body,
        grid=(SH_ROWS // RB,),
        in_specs=[pl.BlockSpec((NC, RB, D), lambda i: (0, i, 0)),
                  pl.BlockSpec((NW, RB), lambda i: (0, i)),
                  pl.BlockSpec((D,), lambda i: (0,)),
                  pl.BlockSpec((D, dl), lambda i: (0, 0)),
                  pl.BlockSpec((D, dr), lambda i: (0, 0))],
        out_specs=[pl.BlockSpec((RB, dl), lambda i: (i, 0)),
                   pl.BlockSpec((RB, dr), lambda i: (i, 0))],
        out_shape=[jax.ShapeDtypeStruct((SH_ROWS, dl), jnp.float32),
                   jax.ShapeDtypeStruct((SH_ROWS, dr), jnp.float32)],
    )(parts, den, b, Wl, Wr)


def _final_tc(parts, den3, mu_b, ls_b, eps):
    lat = mu_b.shape[0]
    D = parts.shape[2]

    def body(p_ref, den_ref, mb_ref, lb_ref, eps_ref, mu_ref, ls_ref, z_ref):
        p = p_ref[0] + p_ref[1]
        ds = jnp.sum(den_ref[...], axis=0)
        mu = p[:, :lat] * (1.0 / (ds[0] + 1e-16))[:, None] \
            + mb_ref[...][None, :]
        ls = p[:, lat:] * (1.0 / (ds[1] + 1e-16))[:, None] \
            + lb_ref[...][None, :]
        mu_ref[...] = mu
        ls_ref[...] = ls
        z_ref[...] = eps_ref[...] * jnp.exp(ls) + mu

    return pl.pallas_call(
        body,
        grid=(SH_ROWS // RB,),
        in_specs=[pl.BlockSpec((NC, RB, D), lambda i: (0, i, 0)),
                  pl.BlockSpec((NW, 2, RB), lambda i: (0, 0, i)),
                  pl.BlockSpec((lat,), lambda i: (0,)),
                  pl.BlockSpec((lat,), lambda i: (0,)),
                  pl.BlockSpec((RB, lat), lambda i: (i, 0))],
        out_specs=[pl.BlockSpec((RB, lat), lambda i: (i, 0)),
                   pl.BlockSpec((RB, lat), lambda i: (i, 0)),
                   pl.BlockSpec((RB, lat), lambda i: (i, 0))],
        out_shape=[jax.ShapeDtypeStruct((SH_ROWS, lat), jnp.float32),
                   jax.ShapeDtypeStruct((SH_ROWS, lat), jnp.float32),
                   jax.ShapeDtypeStruct((SH_ROWS, lat), jnp.float32)],
    )(parts, den3, mu_b, ls_b, eps)


def kernel(x, edge_index, c1_Wl, c1_Wr, c1_att, c1_b, c2_Wl, c2_Wr, c2_att,
           c2_b, mu_Wl, mu_Wr, mu_att, mu_b, ls_Wl, ls_Wr, ls_att, ls_b):
    n = x.shape[0]
    e = edge_index.shape[1]
    loops = jnp.arange(n, dtype=jnp.int32)
    src = jnp.concatenate([edge_index[0], loops])
    dst = jnp.concatenate([edge_index[1], loops])
    e_real = e + n
    e_pad = ((e_real + NW * CH - 1) // (NW * CH)) * (NW * CH)
    src = jnp.concatenate(
        [src, jnp.zeros((e_pad - e_real,), jnp.int32)])
    dst = jnp.concatenate(
        [dst, jnp.full((e_pad - e_real,), n, jnp.int32)])
    sd128 = jnp.stack([src.reshape(-1, 128), dst.reshape(-1, 128)], axis=1)
    sd64 = jnp.stack([src.reshape(-1, 64), dst.reshape(-1, 64)], axis=1)

    xpad = jnp.pad(x, ((0, SH_ROWS - n), (0, 0)))

    xl1, xr1 = _proj_tc(xpad, c1_Wl, c1_Wr)
    p1, d1 = _edge_sc(xl1, xr1, sd128, 0.6 * c1_att, 0.4 * c1_att, 1)
    xl2, xr2 = _merge_proj_tc(p1, d1, c1_b, c2_Wl, c2_Wr)
    p2, d2 = _edge_sc(xl2, xr2, sd64, 0.6 * c2_att, 0.4 * c2_att, 1)
    Wl3 = jnp.concatenate([mu_Wl, ls_Wl], axis=1)
    Wr3 = jnp.concatenate([mu_Wr, ls_Wr], axis=1)
    a3 = jnp.concatenate([mu_att, ls_att])
    xl3, xr3 = _merge_proj_tc(p2, d2, c2_b, Wl3, Wr3)
    p3, d3 = _edge_sc(xl3, xr3, sd128, 0.6 * a3, 0.4 * a3, 2)

    lat = mu_b.shape[0]
    eps = jax.random.normal(jax.random.key(42), (n, lat), jnp.float32)
    epad = jnp.pad(eps, ((0, SH_ROWS - n), (0, 0)))
    muf, lsf, zf = _final_tc(p3, d3.reshape(NW, 2, SH_ROWS),
                             mu_b, ls_b, epad)
    return muf[:n], lsf[:n], zf[:n]


# speedup vs baseline: 18.6974x; 1.0071x over previous
import jax
import jax.numpy as jnp
from jax import lax
from jax.experimental import pallas as pl
from jax.experimental.pallas import tpu as pltpu
from jax.experimental.pallas import tpu_sc as plsc

NC = 2
NS = 16
L = 16
NW = NC * NS
CH = 128
SH_ROWS = 10240
RB = 512


def _vgather(x, idx):
    dnums = lax.GatherDimensionNumbers(
        offset_dims=(), collapsed_slice_dims=(0,), start_index_map=(0,))
    return lax.gather(x, idx[:, None], dnums, slice_sizes=(1,),
                      mode=lax.GatherScatterMode.PROMISE_IN_BOUNDS)


def _edge_sc(xl, xr, sd, att6, att4, heads):
    D = xl.shape[1]
    ch = sd.shape[2]
    kh = D // heads // L
    nchunk = sd.shape[0] // NW
    denw = heads * SH_ROWS
    rpt = SH_ROWS // NS
    ncp = rpt // ch
    mesh = plsc.VectorSubcoreMesh(core_axis_name="c", subcore_axis_name="s")

    def body(xl_hbm, xr_hbm, sd_hbm, a6_hbm, a4_hbm,
             out_hbm, den_hbm,
             acc_sh, gl0, gl1, gr0, gr1, sdv, denv, a6v, a4v,
             sgl0, sgl1, sgr0, sgr1, ssc0, ssc1):
        cid = lax.axis_index("c")
        sid = lax.axis_index("s")
        wid = cid * NS + sid
        zero = jnp.zeros((L,), jnp.float32)
        gl = (gl0, gl1)
        gr = (gr0, gr1)
        sgl = (sgl0, sgl1)
        sgr = (sgr0, sgr1)
        ssc = (ssc0, ssc1)

        def zrow(r, _):
            for k in range(D // L):
                gl0[r, pl.ds(k * L, L)] = zero
            return 0
        lax.fori_loop(0, ch, zrow, 0)

        def zden(i, _):
            denv[pl.ds(i * L, L)] = zero
            return 0
        lax.fori_loop(0, denw // L, zden, 0)

        for j in range(ncp):
            pltpu.sync_copy(gl0, acc_sh.at[pl.ds(sid * rpt + j * ch, ch)])
        pltpu.sync_copy(a6_hbm, a6v)
        pltpu.sync_copy(a4_hbm, a4v)
        plsc.subcore_barrier()

        a6r = [a6v[pl.ds(k * L, L)] for k in range(D // L)]
        a4r = [a4v[pl.ds(k * L, L)] for k in range(D // L)]
        iota = lax.iota(jnp.int32, L)
        shifts = [(iota + s) & (L - 1) for s in (8, 4, 2, 1)]

        def issue(g_idx, b):
            pltpu.sync_copy(sd_hbm.at[g_idx], sdv.at[b])
            pltpu.async_copy(xl_hbm.at[sdv.at[b, 0]], gl[b], sgl[b])
            pltpu.async_copy(xr_hbm.at[sdv.at[b, 1]], gr[b], sgr[b])

        issue(wid * nchunk, 0)

        def half_body(cc, b):
            c = cc * 2 + b
            pltpu.make_async_copy(xl_hbm.at[sdv.at[b, 0]], gl[b],
                                  sgl[b]).wait()
            pltpu.make_async_copy(xr_hbm.at[sdv.at[b, 1]], gr[b],
                                  sgr[b]).wait()

            @pl.when(c + 1 < nchunk)
            def _():
                @pl.when(c >= 1)
                def _():
                    pltpu.make_async_copy(
                        gl[1 - b], acc_sh.at[sdv.at[1 - b, 1]],
                        ssc[1 - b]).wait()
                issue(wid * nchunk + c + 1, 1 - b)

            def group_body(g, _):
                e16s = [zero for _ in range(heads)]
                for j in range(L):
                    e_i = g * L + j
                    for h in range(heads):
                        acc = None
                        vas = []
                        for k in range(kh):
                            col = (h * kh + k) * L
                            va = gl[b][e_i, pl.ds(col, L)]
                            vas.append(va)
                            z = va + gr[b][e_i, pl.ds(col, L)]
                            t = a6r[h * kh + k] * z \
                                + a4r[h * kh + k] * jnp.abs(z)
                            acc = t if acc is None else acc + t
                        for sh in shifts:
                            acc = acc + _vgather(acc, sh)
                        ev = jnp.exp(acc)
                        for k in range(kh):
                            col = (h * kh + k) * L
                            gl[b][e_i, pl.ds(col, L)] = vas[k] * ev
                        e16s[h] = jnp.where(iota == j, ev, e16s[h])
                d16 = sdv[b, 1, pl.ds(g * L, L)]
                for h in range(heads):
                    plsc.addupdate_scatter(denv, [d16 + h * SH_ROWS],
                                           e16s[h])
                return 0
            lax.fori_loop(0, ch // L, group_body, 0)
            pltpu.async_copy(gl[b], acc_sh.at[sdv.at[b, 1]], ssc[b],
                             add=True)

        def chunk_body(cc, _):
            half_body(cc, 0)
            half_body(cc, 1)
            return 0
        lax.fori_loop(0, nchunk // 2, chunk_body, 0)
        pltpu.make_async_copy(gl[0], acc_sh.at[sdv.at[0, 1]], ssc[0]).wait()
        pltpu.make_async_copy(gl[1], acc_sh.at[sdv.at[1, 1]], ssc[1]).wait()

        plsc.subcore_barrier()
        for j in range(ncp):
            r0 = sid * rpt + j * ch
            pltpu.sync_copy(acc_sh.at[pl.ds(r0, ch)],
                            out_hbm.at[cid, pl.ds(r0, ch)])
        pltpu.sync_copy(denv, den_hbm.at[wid])

    f = pl.kernel(
        body,
        out_type=(jax.ShapeDtypeStruct((NC, SH_ROWS, D), jnp.float32),
                  jax.ShapeDtypeStruct((NW, denw), jnp.float32)),
        mesh=mesh,
        compiler_params=pltpu.CompilerParams(use_tc_tiling_on_sc=False,
                                             needs_layout_passes=False),
        scratch_types=[
            pltpu.VMEM_SHARED((SH_ROWS, D), jnp.float32),
            pltpu.VMEM((ch, D), jnp.float32),
            pltpu.VMEM((ch, D), jnp.float32),
            pltpu.VMEM((ch, D), jnp.float32),
            pltpu.VMEM((ch, D), jnp.float32),
            pltpu.VMEM((2, 2, ch), jnp.int32),
            pltpu.VMEM((denw,), jnp.float32),
            pltpu.VMEM((D,), jnp.float32),
            pltpu.VMEM((D,), jnp.float32),
            pltpu.SemaphoreType.DMA,
            pltpu.SemaphoreType.DMA,
            pltpu.SemaphoreType.DMA,
            pltpu.SemaphoreType.DMA,
            pltpu.SemaphoreType.DMA,
            pltpu.SemaphoreType.DMA,
        ],
    )
    return f(xl, xr, sd, att6, att4)


def _proj_tc(h, Wl, Wr):
    din, dl = Wl.shape
    dr = Wr.shape[1]

    def body(h_ref, wl_ref, wr_ref, xl_ref, xr_ref):
        hv = h_ref[...]
        xl_ref[...] = jnp.dot(hv, wl_ref[...],
                              preferred_element_type=jnp.float32)
        xr_ref[...] = jnp.dot(hv, wr_ref[...],
                              preferred_element_type=jnp.float32)

    return pl.pallas_call(
        body,
        grid=(SH_ROWS // RB,),
        in_specs=[pl.BlockSpec((RB, din), lambda i: (i, 0)),
                  pl.BlockSpec((din, dl), lambda i: (0, 0)),
                  pl.BlockSpec((din, dr), lambda i: (0, 0))],
        out_specs=[pl.BlockSpec((RB, dl), lambda i: (i, 0)),
                   pl.BlockSpec((RB, dr), lambda i: (i, 0))],
        out_shape=[jax.ShapeDtypeStruct((SH_ROWS, dl), jnp.float32),
                   jax.ShapeDtypeStruct((SH_ROWS, dr), jnp.float32)],
    )(h, Wl, Wr)


def _merge_proj_tc(parts, den, b, Wl, Wr):
    D = parts.shape[2]
    dl = Wl.shape[1]
    dr = Wr.shape[1]

    def body(p_ref, den_ref, b_ref, wl_ref, wr_ref, xl_ref, xr_ref):
        p = p_ref[0] + p_ref[1]
        dsum = jnp.sum(den_ref[...], axis=0)
        hv = p * (1.0 / (dsum + 1e-16))[:, None] + b_ref[...][None, :]
        hv = jnp.where(hv > 0, hv, jnp.exp(jnp.minimum(hv, 0.0)) - 1.0)
        xl_ref[...] = jnp.dot(hv, wl_ref[...],
                              preferred_element_type=jnp.float32)
        xr_ref[...] = jnp.dot(hv, wr_ref[...],
                              preferred_element_type=jnp.float32)

    return pl.pallas_call(
        body,
        grid=(SH_ROWS // RB,),
        in_specs=[pl.BlockSpec((NC, RB, D), lambda i: (0, i, 0)),
                  pl.BlockSpec((NW, RB), lambda i: (0, i)),
                  pl.BlockSpec((D,), lambda i: (0,)),
                  pl.BlockSpec((D, dl), lambda i: (0, 0)),
                  pl.BlockSpec((D, dr), lambda i: (0, 0))],
        out_specs=[pl.BlockSpec((RB, dl), lambda i: (i, 0)),
                   pl.BlockSpec((RB, dr), lambda i: (i, 0))],
        out_shape=[jax.ShapeDtypeStruct((SH_ROWS, dl), jnp.float32),
                   jax.ShapeDtypeStruct((SH_ROWS, dr), jnp.float32)],
    )(parts, den, b, Wl, Wr)


def _final_tc(parts, den3, mu_b, ls_b, eps):
    lat = mu_b.shape[0]
    D = parts.shape[2]

    def body(p_ref, den_ref, mb_ref, lb_ref, eps_ref, mu_ref, ls_ref, z_ref):
        p = p_ref[0] + p_ref[1]
        ds = jnp.sum(den_ref[...], axis=0)
        mu = p[:, :lat] * (1.0 / (ds[0] + 1e-16))[:, None] \
            + mb_ref[...][None, :]
        ls = p[:, lat:] * (1.0 / (ds[1] + 1e-16))[:, None] \
            + lb_ref[...][None, :]
        mu_ref[...] = mu
        ls_ref[...] = ls
        z_ref[...] = eps_ref[...] * jnp.exp(ls) + mu

    return pl.pallas_call(
        body,
        grid=(SH_ROWS // RB,),
        in_specs=[pl.BlockSpec((NC, RB, D), lambda i: (0, i, 0)),
                  pl.BlockSpec((NW, 2, RB), lambda i: (0, 0, i)),
                  pl.BlockSpec((lat,), lambda i: (0,)),
                  pl.BlockSpec((lat,), lambda i: (0,)),
                  pl.BlockSpec((RB, lat), lambda i: (i, 0))],
        out_specs=[pl.BlockSpec((RB, lat), lambda i: (i, 0)),
                   pl.BlockSpec((RB, lat), lambda i: (i, 0)),
                   pl.BlockSpec((RB, lat), lambda i: (i, 0))],
        out_shape=[jax.ShapeDtypeStruct((SH_ROWS, lat), jnp.float32),
                   jax.ShapeDtypeStruct((SH_ROWS, lat), jnp.float32),
                   jax.ShapeDtypeStruct((SH_ROWS, lat), jnp.float32)],
    )(parts, den3, mu_b, ls_b, eps)


def kernel(x, edge_index, c1_Wl, c1_Wr, c1_att, c1_b, c2_Wl, c2_Wr, c2_att,
           c2_b, mu_Wl, mu_Wr, mu_att, mu_b, ls_Wl, ls_Wr, ls_att, ls_b):
    n = x.shape[0]
    e = edge_index.shape[1]
    loops = jnp.arange(n, dtype=jnp.int32)
    src = jnp.concatenate([edge_index[0], loops])
    dst = jnp.concatenate([edge_index[1], loops])
    e_real = e + n
    e_pad = ((e_real + NW * CH - 1) // (NW * CH)) * (NW * CH)
    src = jnp.concatenate(
        [src, jnp.zeros((e_pad - e_real,), jnp.int32)])
    dst = jnp.concatenate(
        [dst, jnp.full((e_pad - e_real,), n, jnp.int32)])
    sd128 = jnp.stack([src.reshape(-1, 128), dst.reshape(-1, 128)], axis=1)
    sd64 = jnp.stack([src.reshape(-1, 64), dst.reshape(-1, 64)], axis=1)

    xpad = jnp.pad(x, ((0, SH_ROWS - n), (0, 0)))

    xl1, xr1 = _proj_tc(xpad, c1_Wl, c1_Wr)
    p1, d1 = _edge_sc(xl1, xr1, sd128, 0.6 * c1_att, 0.4 * c1_att, 1)
    xl2, xr2 = _merge_proj_tc(p1, d1, c1_b, c2_Wl, c2_Wr)
    p2, d2 = _edge_sc(xl2, xr2, sd64, 0.6 * c2_att, 0.4 * c2_att, 1)
    Wl3 = jnp.concatenate([mu_Wl, ls_Wl], axis=1)
    Wr3 = jnp.concatenate([mu_Wr, ls_Wr], axis=1)
    a3 = jnp.concatenate([mu_att, ls_att])
    xl3, xr3 = _merge_proj_tc(p2, d2, c2_b, Wl3, Wr3)
    p3, d3 = _edge_sc(xl3, xr3, sd128, 0.6 * a3, 0.4 * a3, 2)

    lat = mu_b.shape[0]
    eps = jax.random.normal(jax.random.key(42), (n, lat), jnp.float32)
    epad = jnp.pad(eps, ((0, SH_ROWS - n), (0, 0)))
    muf, lsf, zf = _final_tc(p3, d3.reshape(NW, 2, SH_ROWS),
                             mu_b, ls_b, epad)
    return muf[:n], lsf[:n], zf[:n]
